# Initial kernel scaffold; baseline (speedup 1.0000x reference)
import jax
import jax.numpy as jnp
from jax.experimental import pallas as pl


def kernel(x, edge_index, edge_weight, batch, W1, b1, W2, b2, W3, b3, Wl, bl):
    raise NotImplementedError("write your pallas kernel here")


# speedup vs baseline: 12.8805x; 12.8805x over previous
import functools

import jax
import jax.numpy as jnp
from jax import lax
from jax.experimental import pallas as pl
from jax.experimental.pallas import tpu as pltpu
from jax.experimental.pallas import tpu_sc as plsc

N = 10000
E = 320000
D = 128
H = 128
C = 10
G = 64

NTILES = 32
EDGES_PER_TILE = E // NTILES
KB = 80
NB = EDGES_PER_TILE // KB
ROWS_PER_TILE = 624
NEG = -jnp.inf


def _bcast_lane(v16, e):
    idx = jnp.full((16, 1), e, dtype=jnp.int32)
    dn = lax.GatherDimensionNumbers(
        offset_dims=(), collapsed_slice_dims=(0,), start_index_map=(0,))
    return lax.gather(v16, idx, dn, (1,),
                      mode=lax.GatherScatterMode.PROMISE_IN_BOUNDS)


def _agg_body(h_hbm, src_hbm, dst_hbm, wn_hbm, z_hbm, out_hbm,
              src0, src1, dst0, dst1, wn0, wn1, rows0, rows1,
              acc, s0, s1, si0, si1):
    cid = lax.axis_index("c")
    sid = lax.axis_index("s")
    wid = sid * 2 + cid
    ebase = wid * EDGES_PER_TILE

    row0 = sid * ROWS_PER_TILE
    pltpu.sync_copy(z_hbm, acc.at[pl.ds(row0, ROWS_PER_TILE)])

    @pl.when(sid == 15)
    def _zero_tail():
        pltpu.sync_copy(z_hbm.at[pl.ds(0, 16)],
                        acc.at[pl.ds(16 * ROWS_PER_TILE, 16)])

    def _fire_idx(b, srcb, dstb, wnb, sem):
        base = ebase + b * KB
        pltpu.async_copy(src_hbm.at[pl.ds(base, KB)], srcb, sem)
        pltpu.async_copy(dst_hbm.at[pl.ds(base, KB)], dstb, sem)
        pltpu.async_copy(wn_hbm.at[pl.ds(base, KB)], wnb, sem)

    def _drain_idx(srcb, dstb, wnb, sem):
        pltpu.make_async_copy(src_hbm.at[pl.ds(0, KB)], srcb, sem).wait()
        pltpu.make_async_copy(dst_hbm.at[pl.ds(0, KB)], dstb, sem).wait()
        pltpu.make_async_copy(wn_hbm.at[pl.ds(0, KB)], wnb, sem).wait()

    def _start_gather(srcb, rowsbuf, sem):
        pltpu.async_copy(h_hbm.at[srcb], rowsbuf, sem)

    def _wait_gather(rowsbuf, sem):
        pltpu.make_async_copy(h_hbm.at[src0], rowsbuf, sem).wait()

    def _scale_and_scatter(rowsbuf, wnb, dstb):
        for g in range(5):
            wn16 = wnb[pl.ds(g * 16, 16)]
            for e in range(16):
                r = g * 16 + e
                bc = _bcast_lane(wn16, e)
                for j in range(8):
                    sl = pl.ds(j * 16, 16)
                    rowsbuf[r, sl] = rowsbuf[r, sl] * bc
        pltpu.sync_copy(rowsbuf, acc.at[dstb], add=True)

    plsc.subcore_barrier()

    _fire_idx(0, src0, dst0, wn0, si0)
    _drain_idx(src0, dst0, wn0, si0)
    _start_gather(src0, rows0, s0)
    _fire_idx(1, src1, dst1, wn1, si1)

    def body(i, carry):
        b = 2 * i
        _wait_gather(rows0, s0)
        _drain_idx(src1, dst1, wn1, si1)
        _start_gather(src1, rows1, s1)
        _scale_and_scatter(rows0, wn0, dst0)
        _fire_idx(b + 2, src0, dst0, wn0, si0)
        _wait_gather(rows1, s1)
        _drain_idx(src0, dst0, wn0, si0)
        _start_gather(src0, rows0, s0)
        _scale_and_scatter(rows1, wn1, dst1)

        @pl.when(b + 3 < NB)
        def _fire_next():
            _fire_idx(b + 3, src1, dst1, wn1, si1)

        return carry

    lax.fori_loop(0, (NB - 1) // 2, body, 0)
    _wait_gather(rows0, s0)
    _scale_and_scatter(rows0, wn0, dst0)

    plsc.subcore_barrier()

    pltpu.sync_copy(acc.at[pl.ds(row0, ROWS_PER_TILE)],
                    out_hbm.at[cid, pl.ds(row0, ROWS_PER_TILE)])

    @pl.when(sid == 15)
    def _out_tail():
        pltpu.sync_copy(acc.at[pl.ds(16 * ROWS_PER_TILE, 16)],
                        out_hbm.at[cid, pl.ds(16 * ROWS_PER_TILE, 16)])


def _sc_agg(h, src1d, dst1d, wn1d, zrows):
    mesh = plsc.VectorSubcoreMesh(core_axis_name="c", subcore_axis_name="s")
    f = pl.kernel(
        _agg_body,
        mesh=mesh,
        out_type=jax.ShapeDtypeStruct((2, N, H), jnp.float32),
        scratch_types=[
            pltpu.VMEM((KB,), jnp.int32),
            pltpu.VMEM((KB,), jnp.int32),
            pltpu.VMEM((KB,), jnp.int32),
            pltpu.VMEM((KB,), jnp.int32),
            pltpu.VMEM((KB,), jnp.float32),
            pltpu.VMEM((KB,), jnp.float32),
            pltpu.VMEM((KB, H), jnp.float32),
            pltpu.VMEM((KB, H), jnp.float32),
            pltpu.VMEM_SHARED((N, H), jnp.float32),
            pltpu.SemaphoreType.DMA,
            pltpu.SemaphoreType.DMA,
            pltpu.SemaphoreType.DMA,
            pltpu.SemaphoreType.DMA,
        ],
    )
    return f(h, src1d, dst1d, wn1d, zrows)


def _mm(a, b_mat):
    return lax.dot_general(a, b_mat, (((1,), (1,)), ((), ())),
                           preferred_element_type=jnp.float32,
                           precision=lax.Precision.HIGHEST)


def _k1_body(ew_ref, pdeg_ref, x_ref, w1_ref,
             wn_ref, dinv_ref, t1_ref, hp1_ref):
    ew = ew_ref[...]
    ss = jnp.sum(ew * ew)
    s = 1.0 / jnp.maximum(jnp.sqrt(ss), 1e-12)
    p = pdeg_ref[...]
    deg = s * (p[0] + p[1]) + 1.0
    dinv = lax.rsqrt(deg)
    t1 = _mm(x_ref[...], w1_ref[...])
    wn_ref[...] = ew * s
    dinv_ref[...] = dinv
    t1_ref[...] = t1
    hp1_ref[...] = t1 * dinv


def _k1(ew2d, pdeg, x, w1):
    return pl.pallas_call(
        _k1_body,
        out_shape=(
            jax.ShapeDtypeStruct((E // 128, 128), jnp.float32),
            jax.ShapeDtypeStruct((N, 1), jnp.float32),
            jax.ShapeDtypeStruct((N, H), jnp.float32),
            jax.ShapeDtypeStruct((N, H), jnp.float32),
        ),
    )(ew2d, pdeg, x, w1)


def _k2_body(parts_ref, t_ref, dinv_ref, b_ref, w_ref, tn_ref, hpn_ref):
    p = parts_ref[...]
    dinv = dinv_ref[...]
    h = jnp.maximum(dinv * (p[0] + p[1]) + dinv * dinv * t_ref[...]
                    + b_ref[...], 0.0)
    tn = _mm(h, w_ref[...])
    tn_ref[...] = tn
    hpn_ref[...] = tn * dinv


def _k2(parts, t_prev, dinv, b_prev, w_next):
    return pl.pallas_call(
        _k2_body,
        out_shape=(
            jax.ShapeDtypeStruct((N, H), jnp.float32),
            jax.ShapeDtypeStruct((N, H), jnp.float32),
        ),
    )(parts, t_prev, dinv, b_prev, w_next)


def _k3_body(parts_ref, t_ref, dinv_ref, b_ref, batch_ref, wl_ref, bl_ref,
             out_ref, pooled_ref):
    p = parts_ref[...]
    dinv = dinv_ref[...]
    h = jnp.maximum(dinv * (p[0] + p[1]) + dinv * dinv * t_ref[...]
                    + b_ref[...], 0.0)
    batch = batch_ref[...]

    def body(g, carry):
        m = batch == g
        row = jnp.max(jnp.where(m, h, NEG), axis=0, keepdims=True)
        pooled_ref[pl.ds(g, 1), :] = row
        return carry

    lax.fori_loop(0, G, body, 0)
    out_ref[...] = _mm(pooled_ref[...], wl_ref[...]) + bl_ref[...]


def _k3(parts, t3, dinv, b3, batch2d, wl_pad, bl_pad):
    return pl.pallas_call(
        _k3_body,
        out_shape=jax.ShapeDtypeStruct((G, 16), jnp.float32),
        scratch_shapes=[pltpu.VMEM((G, H), jnp.float32)],
    )(parts, t3, dinv, b3, batch2d, wl_pad, bl_pad)


def kernel(x, edge_index, edge_weight, batch,
           W1, b1, W2, b2, W3, b3, Wl, bl):
    src1d = edge_index[0]
    dst1d = edge_index[1]
    ew = edge_weight.reshape(-1)
    ew2d_tc = ew.reshape(E // 128, 128)
    zrows = jnp.zeros((ROWS_PER_TILE, H), jnp.float32)
    ones_h = jnp.ones((N, H), jnp.float32)
    batch2d = batch.reshape(N, 1)
    b1r = b1.reshape(1, H)
    b2r = b2.reshape(1, H)
    b3r = b3.reshape(1, H)
    wl_pad = jnp.zeros((16, D), jnp.float32).at[:C].set(Wl)
    bl_pad = jnp.zeros((1, 16), jnp.float32).at[0, :C].set(bl)

    pdeg_full = _sc_agg(ones_h, src1d, dst1d, ew, zrows)
    pdeg = pdeg_full[:, :, 0:1]

    wn2d_tc, dinv, t1, hp1 = _k1(ew2d_tc, pdeg, x, W1)
    wn1d = wn2d_tc.reshape(-1)

    parts1 = _sc_agg(hp1, src1d, dst1d, wn1d, zrows)
    t2, hp2 = _k2(parts1, t1, dinv, b1r, W2)
    parts2 = _sc_agg(hp2, src1d, dst1d, wn1d, zrows)
    t3, hp3 = _k2(parts2, t2, dinv, b2r, W3)
    parts3 = _sc_agg(hp3, src1d, dst1d, wn1d, zrows)
    out16 = _k3(parts3, t3, dinv, b3r, batch2d, wl_pad, bl_pad)
    return out16[:, :C]


# speedup vs baseline: 14.5089x; 1.1264x over previous
import functools

import jax
import jax.numpy as jnp
from jax import lax
from jax.experimental import pallas as pl
from jax.experimental.pallas import tpu as pltpu
from jax.experimental.pallas import tpu_sc as plsc

N = 10000
E = 320000
D = 128
H = 128
C = 10
G = 64

NTILES = 32
EDGES_PER_TILE = E // NTILES
KB = 80
NB = EDGES_PER_TILE // KB
ROWS_PER_TILE = 624
NEG = -jnp.inf


def _bcast_lane(v16, e):
    idx = jnp.full((16, 1), e, dtype=jnp.int32)
    dn = lax.GatherDimensionNumbers(
        offset_dims=(), collapsed_slice_dims=(0,), start_index_map=(0,))
    return lax.gather(v16, idx, dn, (1,),
                      mode=lax.GatherScatterMode.PROMISE_IN_BOUNDS)


def _agg_body(h_hbm, src_hbm, dst_hbm, wn_hbm, z_hbm, out_hbm,
              src0, src1, dst0, dst1, wn0, wn1, rows0, rows1,
              acc, s0, s1, si0, si1):
    cid = lax.axis_index("c")
    sid = lax.axis_index("s")
    wid = sid * 2 + cid
    ebase = wid * EDGES_PER_TILE

    row0 = sid * ROWS_PER_TILE
    pltpu.sync_copy(z_hbm, acc.at[pl.ds(row0, ROWS_PER_TILE)])

    @pl.when(sid == 15)
    def _zero_tail():
        pltpu.sync_copy(z_hbm.at[pl.ds(0, 16)],
                        acc.at[pl.ds(16 * ROWS_PER_TILE, 16)])

    def _fire_idx(b, srcb, dstb, wnb, sem):
        base = ebase + b * KB
        pltpu.async_copy(src_hbm.at[pl.ds(base, KB)], srcb, sem)
        pltpu.async_copy(dst_hbm.at[pl.ds(base, KB)], dstb, sem)
        pltpu.async_copy(wn_hbm.at[pl.ds(base, KB)], wnb, sem)

    def _drain_idx(srcb, dstb, wnb, sem):
        pltpu.make_async_copy(src_hbm.at[pl.ds(0, KB)], srcb, sem).wait()
        pltpu.make_async_copy(dst_hbm.at[pl.ds(0, KB)], dstb, sem).wait()
        pltpu.make_async_copy(wn_hbm.at[pl.ds(0, KB)], wnb, sem).wait()

    def _start_gather(srcb, rowsbuf, sem):
        pltpu.async_copy(h_hbm.at[srcb], rowsbuf, sem)

    def _wait_gather(rowsbuf, sem):
        pltpu.make_async_copy(h_hbm.at[src0], rowsbuf, sem).wait()

    def _scale_and_scatter(rowsbuf, wnb, dstb):
        for g in range(5):
            wn16 = wnb[pl.ds(g * 16, 16)]
            for e in range(16):
                r = g * 16 + e
                bc = _bcast_lane(wn16, e)
                for j in range(8):
                    sl = pl.ds(j * 16, 16)
                    rowsbuf[r, sl] = rowsbuf[r, sl] * bc
        pltpu.sync_copy(rowsbuf, acc.at[dstb], add=True)

    plsc.subcore_barrier()

    _fire_idx(0, src0, dst0, wn0, si0)
    _drain_idx(src0, dst0, wn0, si0)
    _start_gather(src0, rows0, s0)
    _fire_idx(1, src1, dst1, wn1, si1)

    def body(i, carry):
        b = 2 * i
        _wait_gather(rows0, s0)
        _drain_idx(src1, dst1, wn1, si1)
        _start_gather(src1, rows1, s1)
        _scale_and_scatter(rows0, wn0, dst0)
        _fire_idx(b + 2, src0, dst0, wn0, si0)
        _wait_gather(rows1, s1)
        _drain_idx(src0, dst0, wn0, si0)
        _start_gather(src0, rows0, s0)
        _scale_and_scatter(rows1, wn1, dst1)

        @pl.when(b + 3 < NB)
        def _fire_next():
            _fire_idx(b + 3, src1, dst1, wn1, si1)

        return carry

    lax.fori_loop(0, (NB - 1) // 2, body, 0)
    _wait_gather(rows0, s0)
    _scale_and_scatter(rows0, wn0, dst0)

    plsc.subcore_barrier()

    pltpu.sync_copy(acc.at[pl.ds(row0, ROWS_PER_TILE)],
                    out_hbm.at[cid, pl.ds(row0, ROWS_PER_TILE)])

    @pl.when(sid == 15)
    def _out_tail():
        pltpu.sync_copy(acc.at[pl.ds(16 * ROWS_PER_TILE, 16)],
                        out_hbm.at[cid, pl.ds(16 * ROWS_PER_TILE, 16)])


def _sc_agg(h, src1d, dst1d, wn1d, zrows):
    mesh = plsc.VectorSubcoreMesh(core_axis_name="c", subcore_axis_name="s")
    f = pl.kernel(
        _agg_body,
        mesh=mesh,
        out_type=jax.ShapeDtypeStruct((2, N, H), jnp.float32),
        scratch_types=[
            pltpu.VMEM((KB,), jnp.int32),
            pltpu.VMEM((KB,), jnp.int32),
            pltpu.VMEM((KB,), jnp.int32),
            pltpu.VMEM((KB,), jnp.int32),
            pltpu.VMEM((KB,), jnp.float32),
            pltpu.VMEM((KB,), jnp.float32),
            pltpu.VMEM((KB, H), jnp.float32),
            pltpu.VMEM((KB, H), jnp.float32),
            pltpu.VMEM_SHARED((N, H), jnp.float32),
            pltpu.SemaphoreType.DMA,
            pltpu.SemaphoreType.DMA,
            pltpu.SemaphoreType.DMA,
            pltpu.SemaphoreType.DMA,
        ],
    )
    return f(h, src1d, dst1d, wn1d, zrows)


def _deg_body(dst_hbm, ew_hbm, z_hbm, out_hbm,
              dst0, dst1, ew0, ew1, val, acc, si0, si1):
    cid = lax.axis_index("c")
    sid = lax.axis_index("s")
    wid = sid * 2 + cid
    ebase = wid * EDGES_PER_TILE

    row0 = sid * ROWS_PER_TILE
    pltpu.sync_copy(z_hbm, acc.at[pl.ds(row0, ROWS_PER_TILE)])

    @pl.when(sid == 15)
    def _zero_tail():
        pltpu.sync_copy(z_hbm.at[pl.ds(0, 16)],
                        acc.at[pl.ds(16 * ROWS_PER_TILE, 16)])

    pltpu.sync_copy(z_hbm.at[pl.ds(0, KB)], val)

    def _fire_idx(b, dstb, ewb, sem):
        base = ebase + b * KB
        pltpu.async_copy(dst_hbm.at[pl.ds(base, KB)], dstb, sem)
        pltpu.async_copy(ew_hbm.at[pl.ds(base, KB)], ewb, sem)

    def _drain_idx(dstb, ewb, sem):
        pltpu.make_async_copy(dst_hbm.at[pl.ds(0, KB)], dstb, sem).wait()
        pltpu.make_async_copy(ew_hbm.at[pl.ds(0, KB)], ewb, sem).wait()

    def _accumulate(dstb, ewb):
        sl = pl.ds(0, 16)
        for g in range(5):
            w16 = ewb[pl.ds(g * 16, 16)]
            for e in range(16):
                r = g * 16 + e
                val[r, sl] = val[r, sl] * 0.0 + _bcast_lane(w16, e)
        pltpu.sync_copy(val, acc.at[dstb], add=True)

    plsc.subcore_barrier()

    _fire_idx(0, dst0, ew0, si0)
    _fire_idx(1, dst1, ew1, si1)

    def body(i, carry):
        b = 2 * i
        _drain_idx(dst0, ew0, si0)
        _accumulate(dst0, ew0)
        _fire_idx(b + 2, dst0, ew0, si0)
        _drain_idx(dst1, ew1, si1)
        _accumulate(dst1, ew1)

        @pl.when(b + 3 < NB)
        def _fire_next():
            _fire_idx(b + 3, dst1, ew1, si1)

        return carry

    lax.fori_loop(0, (NB - 1) // 2, body, 0)
    _drain_idx(dst0, ew0, si0)
    _accumulate(dst0, ew0)

    plsc.subcore_barrier()

    pltpu.sync_copy(acc.at[pl.ds(row0, ROWS_PER_TILE)],
                    out_hbm.at[cid, pl.ds(row0, ROWS_PER_TILE)])

    @pl.when(sid == 15)
    def _out_tail():
        pltpu.sync_copy(acc.at[pl.ds(16 * ROWS_PER_TILE, 16)],
                        out_hbm.at[cid, pl.ds(16 * ROWS_PER_TILE, 16)])


def _sc_deg(dst1d, ew1d, zrows):
    mesh = plsc.VectorSubcoreMesh(core_axis_name="c", subcore_axis_name="s")
    f = pl.kernel(
        _deg_body,
        mesh=mesh,
        out_type=jax.ShapeDtypeStruct((2, N, H), jnp.float32),
        scratch_types=[
            pltpu.VMEM((KB,), jnp.int32),
            pltpu.VMEM((KB,), jnp.int32),
            pltpu.VMEM((KB,), jnp.float32),
            pltpu.VMEM((KB,), jnp.float32),
            pltpu.VMEM((KB, H), jnp.float32),
            pltpu.VMEM_SHARED((N, H), jnp.float32),
            pltpu.SemaphoreType.DMA,
            pltpu.SemaphoreType.DMA,
        ],
    )
    return f(dst1d, ew1d, zrows)


def _mm(a, b_mat):
    return lax.dot_general(a, b_mat, (((1,), (1,)), ((), ())),
                           preferred_element_type=jnp.float32,
                           precision=lax.Precision.HIGHEST)


def _k1_body(ew_ref, pdeg_ref, x_ref, w1_ref,
             wn_ref, dinv_ref, t1_ref, hp1_ref):
    ew = ew_ref[...]
    ss = jnp.sum(ew * ew)
    s = 1.0 / jnp.maximum(jnp.sqrt(ss), 1e-12)
    p = pdeg_ref[...]
    deg = s * (p[0] + p[1]) + 1.0
    dinv = lax.rsqrt(deg)
    t1 = _mm(x_ref[...], w1_ref[...])
    wn_ref[...] = ew * s
    dinv_ref[...] = dinv
    t1_ref[...] = t1
    hp1_ref[...] = t1 * dinv


def _k1(ew2d, pdeg, x, w1):
    return pl.pallas_call(
        _k1_body,
        out_shape=(
            jax.ShapeDtypeStruct((E // 128, 128), jnp.float32),
            jax.ShapeDtypeStruct((N, 1), jnp.float32),
            jax.ShapeDtypeStruct((N, H), jnp.float32),
            jax.ShapeDtypeStruct((N, H), jnp.float32),
        ),
    )(ew2d, pdeg, x, w1)


def _k2_body(parts_ref, t_ref, dinv_ref, b_ref, w_ref, tn_ref, hpn_ref):
    p = parts_ref[...]
    dinv = dinv_ref[...]
    h = jnp.maximum(dinv * (p[0] + p[1]) + dinv * dinv * t_ref[...]
                    + b_ref[...], 0.0)
    tn = _mm(h, w_ref[...])
    tn_ref[...] = tn
    hpn_ref[...] = tn * dinv


def _k2(parts, t_prev, dinv, b_prev, w_next):
    return pl.pallas_call(
        _k2_body,
        out_shape=(
            jax.ShapeDtypeStruct((N, H), jnp.float32),
            jax.ShapeDtypeStruct((N, H), jnp.float32),
        ),
    )(parts, t_prev, dinv, b_prev, w_next)


def _k3_body(parts_ref, t_ref, dinv_ref, b_ref, batch_ref, wl_ref, bl_ref,
             out_ref, pooled_ref):
    p = parts_ref[...]
    dinv = dinv_ref[...]
    h = jnp.maximum(dinv * (p[0] + p[1]) + dinv * dinv * t_ref[...]
                    + b_ref[...], 0.0)
    batch = batch_ref[...]

    def body(g, carry):
        m = batch == g
        row = jnp.max(jnp.where(m, h, NEG), axis=0, keepdims=True)
        pooled_ref[pl.ds(g, 1), :] = row
        return carry

    lax.fori_loop(0, G, body, 0)
    out_ref[...] = _mm(pooled_ref[...], wl_ref[...]) + bl_ref[...]


def _k3(parts, t3, dinv, b3, batch2d, wl_pad, bl_pad):
    return pl.pallas_call(
        _k3_body,
        out_shape=jax.ShapeDtypeStruct((G, 16), jnp.float32),
        scratch_shapes=[pltpu.VMEM((G, H), jnp.float32)],
    )(parts, t3, dinv, b3, batch2d, wl_pad, bl_pad)


def kernel(x, edge_index, edge_weight, batch,
           W1, b1, W2, b2, W3, b3, Wl, bl):
    src1d = edge_index[0]
    dst1d = edge_index[1]
    ew = edge_weight.reshape(-1)
    ew2d_tc = ew.reshape(E // 128, 128)
    zrows = jnp.zeros((ROWS_PER_TILE, H), jnp.float32)
    batch2d = batch.reshape(N, 1)
    b1r = b1.reshape(1, H)
    b2r = b2.reshape(1, H)
    b3r = b3.reshape(1, H)
    wl_pad = jnp.zeros((16, D), jnp.float32).at[:C].set(Wl)
    bl_pad = jnp.zeros((1, 16), jnp.float32).at[0, :C].set(bl)

    pdeg_full = _sc_deg(dst1d, ew, zrows)
    pdeg = pdeg_full[:, :, 0:1]

    wn2d_tc, dinv, t1, hp1 = _k1(ew2d_tc, pdeg, x, W1)
    wn1d = wn2d_tc.reshape(-1)

    parts1 = _sc_agg(hp1, src1d, dst1d, wn1d, zrows)
    t2, hp2 = _k2(parts1, t1, dinv, b1r, W2)
    parts2 = _sc_agg(hp2, src1d, dst1d, wn1d, zrows)
    t3, hp3 = _k2(parts2, t2, dinv, b2r, W3)
    parts3 = _sc_agg(hp3, src1d, dst1d, wn1d, zrows)
    out16 = _k3(parts3, t3, dinv, b3r, batch2d, wl_pad, bl_pad)
    return out16[:, :C]


# speedup vs baseline: 16.0869x; 1.1088x over previous
import functools

import jax
import jax.numpy as jnp
from jax import lax
from jax.experimental import pallas as pl
from jax.experimental.pallas import tpu as pltpu
from jax.experimental.pallas import tpu_sc as plsc

N = 10000
E = 320000
D = 128
H = 128
C = 10
G = 64

NTILES = 32
EDGES_PER_TILE = E // NTILES
KB = 80
NB = EDGES_PER_TILE // KB
ROWS_PER_TILE = 624
NEG = -jnp.inf


def _bcast_lane(v16, e):
    idx = jnp.full((16, 1), e, dtype=jnp.int32)
    dn = lax.GatherDimensionNumbers(
        offset_dims=(), collapsed_slice_dims=(0,), start_index_map=(0,))
    return lax.gather(v16, idx, dn, (1,),
                      mode=lax.GatherScatterMode.PROMISE_IN_BOUNDS)


def _agg_body(h_hbm, src_hbm, dst_hbm, wn_hbm, z_hbm, out_hbm,
              src0, src1, dst0, dst1, wn0, wn1, dsc0, dsc1, rows0, rows1,
              acc, s0, s1, si0, si1, sc0, sc1):
    cid = lax.axis_index("c")
    sid = lax.axis_index("s")
    wid = sid * 2 + cid
    ebase = wid * EDGES_PER_TILE

    row0 = sid * ROWS_PER_TILE
    pltpu.sync_copy(z_hbm, acc.at[pl.ds(row0, ROWS_PER_TILE)])

    @pl.when(sid == 15)
    def _zero_tail():
        pltpu.sync_copy(z_hbm.at[pl.ds(0, 16)],
                        acc.at[pl.ds(16 * ROWS_PER_TILE, 16)])

    def _fire_idx(b, srcb, dstb, wnb, sem):
        base = ebase + b * KB
        pltpu.async_copy(src_hbm.at[pl.ds(base, KB)], srcb, sem)
        pltpu.async_copy(dst_hbm.at[pl.ds(base, KB)], dstb, sem)
        pltpu.async_copy(wn_hbm.at[pl.ds(base, KB)], wnb, sem)

    def _drain_idx(srcb, dstb, wnb, sem):
        pltpu.make_async_copy(src_hbm.at[pl.ds(0, KB)], srcb, sem).wait()
        pltpu.make_async_copy(dst_hbm.at[pl.ds(0, KB)], dstb, sem).wait()
        pltpu.make_async_copy(wn_hbm.at[pl.ds(0, KB)], wnb, sem).wait()

    def _start_gather(srcb, rowsbuf, sem):
        pltpu.async_copy(h_hbm.at[srcb], rowsbuf, sem)

    def _wait_gather(rowsbuf, sem):
        pltpu.make_async_copy(h_hbm.at[src0], rowsbuf, sem).wait()

    def _scale(rowsbuf, wnb):
        for g in range(5):
            wn16 = wnb[pl.ds(g * 16, 16)]
            for e in range(16):
                r = g * 16 + e
                bc = _bcast_lane(wn16, e)
                for j in range(8):
                    sl = pl.ds(j * 16, 16)
                    rowsbuf[r, sl] = rowsbuf[r, sl] * bc

    def _copy_idx(srcb, dstb):
        for g in range(5):
            sl = pl.ds(g * 16, 16)
            dstb[sl] = srcb[sl]

    def _scatter_async(rowsbuf, dscb, sem):
        pltpu.async_copy(rowsbuf, acc.at[dscb], sem, add=True)

    def _wait_scatter(rowsbuf, dscb, sem):
        pltpu.make_async_copy(rowsbuf, acc.at[dscb], sem).wait()

    plsc.subcore_barrier()

    _fire_idx(0, src0, dst0, wn0, si0)
    _drain_idx(src0, dst0, wn0, si0)
    _start_gather(src0, rows0, s0)
    _fire_idx(1, src1, dst1, wn1, si1)

    def body(i, carry):
        b = 2 * i
        _wait_gather(rows0, s0)

        @pl.when(i > 0)
        def _w1():
            _wait_scatter(rows1, dsc1, sc1)

        _drain_idx(src1, dst1, wn1, si1)
        _start_gather(src1, rows1, s1)
        _scale(rows0, wn0)
        _copy_idx(dst0, dsc0)
        _scatter_async(rows0, dsc0, sc0)
        _fire_idx(b + 2, src0, dst0, wn0, si0)
        _wait_gather(rows1, s1)
        _wait_scatter(rows0, dsc0, sc0)
        _drain_idx(src0, dst0, wn0, si0)
        _start_gather(src0, rows0, s0)
        _scale(rows1, wn1)
        _copy_idx(dst1, dsc1)
        _scatter_async(rows1, dsc1, sc1)

        @pl.when(b + 3 < NB)
        def _fire_next():
            _fire_idx(b + 3, src1, dst1, wn1, si1)

        return carry

    lax.fori_loop(0, (NB - 1) // 2, body, 0)
    _wait_gather(rows0, s0)
    _scale(rows0, wn0)
    pltpu.sync_copy(rows0, acc.at[dst0], add=True)
    _wait_scatter(rows1, dsc1, sc1)

    plsc.subcore_barrier()

    pltpu.sync_copy(acc.at[pl.ds(row0, ROWS_PER_TILE)],
                    out_hbm.at[cid, pl.ds(row0, ROWS_PER_TILE)])

    @pl.when(sid == 15)
    def _out_tail():
        pltpu.sync_copy(acc.at[pl.ds(16 * ROWS_PER_TILE, 16)],
                        out_hbm.at[cid, pl.ds(16 * ROWS_PER_TILE, 16)])


def _sc_agg(h, src1d, dst1d, wn1d, zrows):
    mesh = plsc.VectorSubcoreMesh(core_axis_name="c", subcore_axis_name="s")
    f = pl.kernel(
        _agg_body,
        mesh=mesh,
        out_type=jax.ShapeDtypeStruct((2, N, H), jnp.float32),
        scratch_types=[
            pltpu.VMEM((KB,), jnp.int32),
            pltpu.VMEM((KB,), jnp.int32),
            pltpu.VMEM((KB,), jnp.int32),
            pltpu.VMEM((KB,), jnp.int32),
            pltpu.VMEM((KB,), jnp.float32),
            pltpu.VMEM((KB,), jnp.float32),
            pltpu.VMEM((KB,), jnp.int32),
            pltpu.VMEM((KB,), jnp.int32),
            pltpu.VMEM((KB, H), jnp.float32),
            pltpu.VMEM((KB, H), jnp.float32),
            pltpu.VMEM_SHARED((N, H), jnp.float32),
            pltpu.SemaphoreType.DMA,
            pltpu.SemaphoreType.DMA,
            pltpu.SemaphoreType.DMA,
            pltpu.SemaphoreType.DMA,
            pltpu.SemaphoreType.DMA,
            pltpu.SemaphoreType.DMA,
        ],
    )
    return f(h, src1d, dst1d, wn1d, zrows)


def _deg_body(dst_hbm, ew_hbm, z_hbm, out_hbm,
              dst0, dst1, ew0, ew1, val, acc, si0, si1):
    cid = lax.axis_index("c")
    sid = lax.axis_index("s")
    wid = sid * 2 + cid
    ebase = wid * EDGES_PER_TILE

    row0 = sid * ROWS_PER_TILE
    pltpu.sync_copy(z_hbm, acc.at[pl.ds(row0, ROWS_PER_TILE)])

    @pl.when(sid == 15)
    def _zero_tail():
        pltpu.sync_copy(z_hbm.at[pl.ds(0, 16)],
                        acc.at[pl.ds(16 * ROWS_PER_TILE, 16)])

    pltpu.sync_copy(z_hbm.at[pl.ds(0, KB)], val)

    def _fire_idx(b, dstb, ewb, sem):
        base = ebase + b * KB
        pltpu.async_copy(dst_hbm.at[pl.ds(base, KB)], dstb, sem)
        pltpu.async_copy(ew_hbm.at[pl.ds(base, KB)], ewb, sem)

    def _drain_idx(dstb, ewb, sem):
        pltpu.make_async_copy(dst_hbm.at[pl.ds(0, KB)], dstb, sem).wait()
        pltpu.make_async_copy(ew_hbm.at[pl.ds(0, KB)], ewb, sem).wait()

    def _accumulate(dstb, ewb):
        sl = pl.ds(0, 16)
        for g in range(5):
            w16 = ewb[pl.ds(g * 16, 16)]
            for e in range(16):
                r = g * 16 + e
                val[r, sl] = val[r, sl] * 0.0 + _bcast_lane(w16, e)
        pltpu.sync_copy(val, acc.at[dstb], add=True)

    plsc.subcore_barrier()

    _fire_idx(0, dst0, ew0, si0)
    _fire_idx(1, dst1, ew1, si1)

    def body(i, carry):
        b = 2 * i
        _drain_idx(dst0, ew0, si0)
        _accumulate(dst0, ew0)
        _fire_idx(b + 2, dst0, ew0, si0)
        _drain_idx(dst1, ew1, si1)
        _accumulate(dst1, ew1)

        @pl.when(b + 3 < NB)
        def _fire_next():
            _fire_idx(b + 3, dst1, ew1, si1)

        return carry

    lax.fori_loop(0, (NB - 1) // 2, body, 0)
    _drain_idx(dst0, ew0, si0)
    _accumulate(dst0, ew0)

    plsc.subcore_barrier()

    pltpu.sync_copy(acc.at[pl.ds(row0, ROWS_PER_TILE)],
                    out_hbm.at[cid, pl.ds(row0, ROWS_PER_TILE)])

    @pl.when(sid == 15)
    def _out_tail():
        pltpu.sync_copy(acc.at[pl.ds(16 * ROWS_PER_TILE, 16)],
                        out_hbm.at[cid, pl.ds(16 * ROWS_PER_TILE, 16)])


def _sc_deg(dst1d, ew1d, zrows):
    mesh = plsc.VectorSubcoreMesh(core_axis_name="c", subcore_axis_name="s")
    f = pl.kernel(
        _deg_body,
        mesh=mesh,
        out_type=jax.ShapeDtypeStruct((2, N, H), jnp.float32),
        scratch_types=[
            pltpu.VMEM((KB,), jnp.int32),
            pltpu.VMEM((KB,), jnp.int32),
            pltpu.VMEM((KB,), jnp.float32),
            pltpu.VMEM((KB,), jnp.float32),
            pltpu.VMEM((KB, H), jnp.float32),
            pltpu.VMEM_SHARED((N, H), jnp.float32),
            pltpu.SemaphoreType.DMA,
            pltpu.SemaphoreType.DMA,
        ],
    )
    return f(dst1d, ew1d, zrows)


def _mm(a, b_mat):
    return lax.dot_general(a, b_mat, (((1,), (1,)), ((), ())),
                           preferred_element_type=jnp.float32,
                           precision=lax.Precision.HIGHEST)


def _k1_body(ew_ref, pdeg_ref, x_ref, w1_ref,
             wn_ref, dinv_ref, t1_ref, hp1_ref):
    ew = ew_ref[...]
    ss = jnp.sum(ew * ew)
    s = 1.0 / jnp.maximum(jnp.sqrt(ss), 1e-12)
    p = pdeg_ref[...]
    deg = s * (p[0] + p[1]) + 1.0
    dinv = lax.rsqrt(deg)
    t1 = _mm(x_ref[...], w1_ref[...])
    wn_ref[...] = ew * s
    dinv_ref[...] = dinv
    t1_ref[...] = t1
    hp1_ref[...] = t1 * dinv


def _k1(ew2d, pdeg, x, w1):
    return pl.pallas_call(
        _k1_body,
        out_shape=(
            jax.ShapeDtypeStruct((E // 128, 128), jnp.float32),
            jax.ShapeDtypeStruct((N, 1), jnp.float32),
            jax.ShapeDtypeStruct((N, H), jnp.float32),
            jax.ShapeDtypeStruct((N, H), jnp.float32),
        ),
    )(ew2d, pdeg, x, w1)


def _k2_body(parts_ref, t_ref, dinv_ref, b_ref, w_ref, tn_ref, hpn_ref):
    p = parts_ref[...]
    dinv = dinv_ref[...]
    h = jnp.maximum(dinv * (p[0] + p[1]) + dinv * dinv * t_ref[...]
                    + b_ref[...], 0.0)
    tn = _mm(h, w_ref[...])
    tn_ref[...] = tn
    hpn_ref[...] = tn * dinv


def _k2(parts, t_prev, dinv, b_prev, w_next):
    return pl.pallas_call(
        _k2_body,
        out_shape=(
            jax.ShapeDtypeStruct((N, H), jnp.float32),
            jax.ShapeDtypeStruct((N, H), jnp.float32),
        ),
    )(parts, t_prev, dinv, b_prev, w_next)


def _k3_body(parts_ref, t_ref, dinv_ref, b_ref, batch_ref, wl_ref, bl_ref,
             out_ref, pooled_ref):
    p = parts_ref[...]
    dinv = dinv_ref[...]
    h = jnp.maximum(dinv * (p[0] + p[1]) + dinv * dinv * t_ref[...]
                    + b_ref[...], 0.0)
    batch = batch_ref[...]

    def body(g, carry):
        m = batch == g
        row = jnp.max(jnp.where(m, h, NEG), axis=0, keepdims=True)
        pooled_ref[pl.ds(g, 1), :] = row
        return carry

    lax.fori_loop(0, G, body, 0)
    out_ref[...] = _mm(pooled_ref[...], wl_ref[...]) + bl_ref[...]


def _k3(parts, t3, dinv, b3, batch2d, wl_pad, bl_pad):
    return pl.pallas_call(
        _k3_body,
        out_shape=jax.ShapeDtypeStruct((G, 16), jnp.float32),
        scratch_shapes=[pltpu.VMEM((G, H), jnp.float32)],
    )(parts, t3, dinv, b3, batch2d, wl_pad, bl_pad)


def kernel(x, edge_index, edge_weight, batch,
           W1, b1, W2, b2, W3, b3, Wl, bl):
    src1d = edge_index[0]
    dst1d = edge_index[1]
    ew = edge_weight.reshape(-1)
    ew2d_tc = ew.reshape(E // 128, 128)
    zrows = jnp.zeros((ROWS_PER_TILE, H), jnp.float32)
    batch2d = batch.reshape(N, 1)
    b1r = b1.reshape(1, H)
    b2r = b2.reshape(1, H)
    b3r = b3.reshape(1, H)
    wl_pad = jnp.zeros((16, D), jnp.float32).at[:C].set(Wl)
    bl_pad = jnp.zeros((1, 16), jnp.float32).at[0, :C].set(bl)

    pdeg_full = _sc_deg(dst1d, ew, zrows)
    pdeg = pdeg_full[:, :, 0:1]

    wn2d_tc, dinv, t1, hp1 = _k1(ew2d_tc, pdeg, x, W1)
    wn1d = wn2d_tc.reshape(-1)

    parts1 = _sc_agg(hp1, src1d, dst1d, wn1d, zrows)
    t2, hp2 = _k2(parts1, t1, dinv, b1r, W2)
    parts2 = _sc_agg(hp2, src1d, dst1d, wn1d, zrows)
    t3, hp3 = _k2(parts2, t2, dinv, b2r, W3)
    parts3 = _sc_agg(hp3, src1d, dst1d, wn1d, zrows)
    out16 = _k3(parts3, t3, dinv, b3r, batch2d, wl_pad, bl_pad)
    return out16[:, :C]


# speedup vs baseline: 17.9093x; 1.1133x over previous
import functools

import jax
import jax.numpy as jnp
from jax import lax
from jax.experimental import pallas as pl
from jax.experimental.pallas import tpu as pltpu
from jax.experimental.pallas import tpu_sc as plsc

N = 10000
E = 320000
D = 128
H = 128
C = 10
G = 64

NTILES = 32
EDGES_PER_TILE = E // NTILES
KB = 80
NB = EDGES_PER_TILE // KB
ROWS_PER_TILE = 624
NEG = -jnp.inf


def _bcast_lane(v16, e):
    idx = jnp.full((16, 1), e, dtype=jnp.int32)
    dn = lax.GatherDimensionNumbers(
        offset_dims=(), collapsed_slice_dims=(0,), start_index_map=(0,))
    return lax.gather(v16, idx, dn, (1,),
                      mode=lax.GatherScatterMode.PROMISE_IN_BOUNDS)


def _agg_body(h_hbm, src_hbm, dst_hbm, wn_hbm, z_hbm, out_hbm,
              src0, src1, dst0, dst1, wn0, wn1, dsc0, dsc1, rows0, rows1,
              acc, s0, s1, si0, si1, sc0, sc1):
    cid = lax.axis_index("c")
    sid = lax.axis_index("s")
    wid = sid * 2 + cid
    ebase = wid * EDGES_PER_TILE

    row0 = sid * ROWS_PER_TILE
    pltpu.sync_copy(z_hbm, acc.at[pl.ds(row0, ROWS_PER_TILE)])

    @pl.when(sid == 15)
    def _zero_tail():
        pltpu.sync_copy(z_hbm.at[pl.ds(0, 16)],
                        acc.at[pl.ds(16 * ROWS_PER_TILE, 16)])

    def _fire_idx(b, srcb, dstb, wnb, sem):
        base = ebase + b * KB
        pltpu.async_copy(src_hbm.at[pl.ds(base, KB)], srcb, sem)
        pltpu.async_copy(dst_hbm.at[pl.ds(base, KB)], dstb, sem)
        pltpu.async_copy(wn_hbm.at[pl.ds(base, KB)], wnb, sem)

    def _drain_idx(srcb, dstb, wnb, sem):
        pltpu.make_async_copy(src_hbm.at[pl.ds(0, KB)], srcb, sem).wait()
        pltpu.make_async_copy(dst_hbm.at[pl.ds(0, KB)], dstb, sem).wait()
        pltpu.make_async_copy(wn_hbm.at[pl.ds(0, KB)], wnb, sem).wait()

    def _start_gather(srcb, rowsbuf, sem):
        pltpu.async_copy(h_hbm.at[srcb], rowsbuf, sem)

    def _wait_gather(rowsbuf, sem):
        pltpu.make_async_copy(h_hbm.at[src0], rowsbuf, sem).wait()

    def _scale(rowsbuf, wnb):
        for g in range(5):
            wn16 = wnb[pl.ds(g * 16, 16)]
            for e in range(16):
                r = g * 16 + e
                bc = _bcast_lane(wn16, e)
                for j in range(8):
                    sl = pl.ds(j * 16, 16)
                    rowsbuf[r, sl] = rowsbuf[r, sl] * bc

    def _copy_idx(srcb, dstb):
        for g in range(5):
            sl = pl.ds(g * 16, 16)
            dstb[sl] = srcb[sl]

    def _scatter_async(rowsbuf, dscb, sem):
        pltpu.async_copy(rowsbuf, acc.at[dscb], sem, add=True)

    def _wait_scatter(rowsbuf, dscb, sem):
        pltpu.make_async_copy(rowsbuf, acc.at[dscb], sem).wait()

    plsc.subcore_barrier()

    _fire_idx(0, src0, dst0, wn0, si0)
    _drain_idx(src0, dst0, wn0, si0)
    _start_gather(src0, rows0, s0)
    _fire_idx(1, src1, dst1, wn1, si1)

    def body(i, carry):
        b = 2 * i
        _wait_gather(rows0, s0)

        @pl.when(i > 0)
        def _w1():
            _wait_scatter(rows1, dsc1, sc1)

        _drain_idx(src1, dst1, wn1, si1)
        _start_gather(src1, rows1, s1)
        _scale(rows0, wn0)
        _copy_idx(dst0, dsc0)
        _scatter_async(rows0, dsc0, sc0)
        _fire_idx(b + 2, src0, dst0, wn0, si0)
        _wait_gather(rows1, s1)
        _wait_scatter(rows0, dsc0, sc0)
        _drain_idx(src0, dst0, wn0, si0)
        _start_gather(src0, rows0, s0)
        _scale(rows1, wn1)
        _copy_idx(dst1, dsc1)
        _scatter_async(rows1, dsc1, sc1)

        @pl.when(b + 3 < NB)
        def _fire_next():
            _fire_idx(b + 3, src1, dst1, wn1, si1)

        return carry

    lax.fori_loop(0, (NB - 1) // 2, body, 0)
    _wait_gather(rows0, s0)
    _scale(rows0, wn0)
    pltpu.sync_copy(rows0, acc.at[dst0], add=True)
    _wait_scatter(rows1, dsc1, sc1)

    plsc.subcore_barrier()

    pltpu.sync_copy(acc.at[pl.ds(row0, ROWS_PER_TILE)],
                    out_hbm.at[cid, pl.ds(row0, ROWS_PER_TILE)])

    @pl.when(sid == 15)
    def _out_tail():
        pltpu.sync_copy(acc.at[pl.ds(16 * ROWS_PER_TILE, 16)],
                        out_hbm.at[cid, pl.ds(16 * ROWS_PER_TILE, 16)])


def _sc_agg(h, src1d, dst1d, wn1d, zrows):
    mesh = plsc.VectorSubcoreMesh(core_axis_name="c", subcore_axis_name="s")
    f = pl.kernel(
        _agg_body,
        mesh=mesh,
        out_type=jax.ShapeDtypeStruct((2, N, H), jnp.float32),
        scratch_types=[
            pltpu.VMEM((KB,), jnp.int32),
            pltpu.VMEM((KB,), jnp.int32),
            pltpu.VMEM((KB,), jnp.int32),
            pltpu.VMEM((KB,), jnp.int32),
            pltpu.VMEM((KB,), jnp.float32),
            pltpu.VMEM((KB,), jnp.float32),
            pltpu.VMEM((KB,), jnp.int32),
            pltpu.VMEM((KB,), jnp.int32),
            pltpu.VMEM((KB, H), jnp.float32),
            pltpu.VMEM((KB, H), jnp.float32),
            pltpu.VMEM_SHARED((N, H), jnp.float32),
            pltpu.SemaphoreType.DMA,
            pltpu.SemaphoreType.DMA,
            pltpu.SemaphoreType.DMA,
            pltpu.SemaphoreType.DMA,
            pltpu.SemaphoreType.DMA,
            pltpu.SemaphoreType.DMA,
        ],
    )
    return f(h, src1d, dst1d, wn1d, zrows)


def _deg_body(dst_hbm, ew_hbm, z_hbm, out_hbm,
              dst0, dst1, ew0, ew1, val, acc, si0, si1):
    cid = lax.axis_index("c")
    sid = lax.axis_index("s")
    wid = sid * 2 + cid
    ebase = wid * EDGES_PER_TILE

    row0 = sid * ROWS_PER_TILE
    pltpu.sync_copy(z_hbm, acc.at[pl.ds(row0, ROWS_PER_TILE)])

    @pl.when(sid == 15)
    def _zero_tail():
        pltpu.sync_copy(z_hbm.at[pl.ds(0, 16)],
                        acc.at[pl.ds(16 * ROWS_PER_TILE, 16)])

    pltpu.sync_copy(z_hbm.at[pl.ds(0, KB)], val)

    def _fire_idx(b, dstb, ewb, sem):
        base = ebase + b * KB
        pltpu.async_copy(dst_hbm.at[pl.ds(base, KB)], dstb, sem)
        pltpu.async_copy(ew_hbm.at[pl.ds(base, KB)], ewb, sem)

    def _drain_idx(dstb, ewb, sem):
        pltpu.make_async_copy(dst_hbm.at[pl.ds(0, KB)], dstb, sem).wait()
        pltpu.make_async_copy(ew_hbm.at[pl.ds(0, KB)], ewb, sem).wait()

    def _accumulate(dstb, ewb):
        sl = pl.ds(0, 16)
        for g in range(5):
            w16 = ewb[pl.ds(g * 16, 16)]
            for e in range(16):
                r = g * 16 + e
                val[r, sl] = val[r, sl] * 0.0 + _bcast_lane(w16, e)
        pltpu.sync_copy(val, acc.at[dstb], add=True)

    plsc.subcore_barrier()

    _fire_idx(0, dst0, ew0, si0)
    _fire_idx(1, dst1, ew1, si1)

    def body(i, carry):
        b = 2 * i
        _drain_idx(dst0, ew0, si0)
        _accumulate(dst0, ew0)
        _fire_idx(b + 2, dst0, ew0, si0)
        _drain_idx(dst1, ew1, si1)
        _accumulate(dst1, ew1)

        @pl.when(b + 3 < NB)
        def _fire_next():
            _fire_idx(b + 3, dst1, ew1, si1)

        return carry

    lax.fori_loop(0, (NB - 1) // 2, body, 0)
    _drain_idx(dst0, ew0, si0)
    _accumulate(dst0, ew0)

    plsc.subcore_barrier()

    pltpu.sync_copy(acc.at[pl.ds(row0, ROWS_PER_TILE)],
                    out_hbm.at[cid, pl.ds(row0, ROWS_PER_TILE)])

    @pl.when(sid == 15)
    def _out_tail():
        pltpu.sync_copy(acc.at[pl.ds(16 * ROWS_PER_TILE, 16)],
                        out_hbm.at[cid, pl.ds(16 * ROWS_PER_TILE, 16)])


def _sc_deg(dst1d, ew1d, zrows):
    mesh = plsc.VectorSubcoreMesh(core_axis_name="c", subcore_axis_name="s")
    f = pl.kernel(
        _deg_body,
        mesh=mesh,
        out_type=jax.ShapeDtypeStruct((2, N, H), jnp.float32),
        scratch_types=[
            pltpu.VMEM((KB,), jnp.int32),
            pltpu.VMEM((KB,), jnp.int32),
            pltpu.VMEM((KB,), jnp.float32),
            pltpu.VMEM((KB,), jnp.float32),
            pltpu.VMEM((KB, H), jnp.float32),
            pltpu.VMEM_SHARED((N, H), jnp.float32),
            pltpu.SemaphoreType.DMA,
            pltpu.SemaphoreType.DMA,
        ],
    )
    return f(dst1d, ew1d, zrows)


NEGF = float("-inf")
CH = 64


def _pool_body(h_hbm, starts_hbm, z_hbm, out_hbm, stbuf, chunk, rowbuf):
    cid = lax.axis_index("c")
    sid = lax.axis_index("s")
    wid = sid * 2 + cid
    g0 = 2 * wid

    pltpu.sync_copy(starts_hbm, stbuf)
    v3 = stbuf[pl.ds(g0, 16)]
    s0 = v3[0]
    s1 = v3[1]
    e1 = v3[2]

    negv = jnp.full((16,), NEGF, jnp.float32)

    for k in range(2):
        sg = s0 if k == 0 else s1
        eg = s1 if k == 0 else e1
        sal = (sg // 8) * 8
        nch = (eg - sal + CH - 1) // CH

        def body(c, accs, sg=sg, eg=eg, sal=sal):
            off = sal + c * CH
            offc = pl.multiple_of(jnp.minimum(off, N - CH), 8)
            pltpu.sync_copy(h_hbm.at[pl.ds(offc, CH)], chunk)
            accs = list(accs)
            for r in range(CH):
                rid = offc + r
                valid = (rid >= sg) & (rid < eg)
                for j in range(8):
                    cur = chunk[r, pl.ds(j * 16, 16)]
                    sel = jnp.where(valid, cur, negv)
                    accs[j] = jnp.maximum(accs[j], sel)
            return tuple(accs)

        accs = lax.fori_loop(0, nch, body, tuple([negv] * 8))

        pltpu.sync_copy(z_hbm.at[pl.ds(0, 1)], rowbuf)
        for j in range(8):
            sl = pl.ds(j * 16, 16)
            rowbuf[0, sl] = rowbuf[0, sl] * 0.0 + accs[j]
        pltpu.sync_copy(rowbuf, out_hbm.at[g0 + k])


def _sc_pool(h3, starts1d, zrows):
    mesh = plsc.VectorSubcoreMesh(core_axis_name="c", subcore_axis_name="s")
    f = pl.kernel(
        _pool_body,
        mesh=mesh,
        out_type=jax.ShapeDtypeStruct((G, 1, H), jnp.float32),
        scratch_types=[
            pltpu.VMEM((80,), jnp.int32),
            pltpu.VMEM((CH, H), jnp.float32),
            pltpu.VMEM((1, H), jnp.float32),
        ],
    )
    return f(h3, starts1d, zrows)


def _mm(a, b_mat):
    return lax.dot_general(a, b_mat, (((1,), (1,)), ((), ())),
                           preferred_element_type=jnp.float32,
                           precision=lax.Precision.HIGHEST)


def _k1_body(ew_ref, pdeg_ref, x_ref, w1_ref,
             wn_ref, dinv_ref, t1_ref, hp1_ref):
    ew = ew_ref[...]
    ss = jnp.sum(ew * ew)
    s = 1.0 / jnp.maximum(jnp.sqrt(ss), 1e-12)
    p = pdeg_ref[...]
    deg = s * (p[0] + p[1]) + 1.0
    dinv = lax.rsqrt(deg)
    t1 = _mm(x_ref[...], w1_ref[...])
    wn_ref[...] = ew * s
    dinv_ref[...] = dinv
    t1_ref[...] = t1
    hp1_ref[...] = t1 * dinv


def _k1(ew2d, pdeg, x, w1):
    return pl.pallas_call(
        _k1_body,
        out_shape=(
            jax.ShapeDtypeStruct((E // 128, 128), jnp.float32),
            jax.ShapeDtypeStruct((N, 1), jnp.float32),
            jax.ShapeDtypeStruct((N, H), jnp.float32),
            jax.ShapeDtypeStruct((N, H), jnp.float32),
        ),
    )(ew2d, pdeg, x, w1)


def _k2_body(parts_ref, t_ref, dinv_ref, b_ref, w_ref, tn_ref, hpn_ref):
    p = parts_ref[...]
    dinv = dinv_ref[...]
    h = jnp.maximum(dinv * (p[0] + p[1]) + dinv * dinv * t_ref[...]
                    + b_ref[...], 0.0)
    tn = _mm(h, w_ref[...])
    tn_ref[...] = tn
    hpn_ref[...] = tn * dinv


def _k2(parts, t_prev, dinv, b_prev, w_next):
    return pl.pallas_call(
        _k2_body,
        out_shape=(
            jax.ShapeDtypeStruct((N, H), jnp.float32),
            jax.ShapeDtypeStruct((N, H), jnp.float32),
        ),
    )(parts, t_prev, dinv, b_prev, w_next)


def _k3a_body(parts_ref, t_ref, dinv_ref, b_ref, batch_ref, h_ref,
              starts_ref):
    p = parts_ref[...]
    dinv = dinv_ref[...]
    h_ref[...] = jnp.maximum(dinv * (p[0] + p[1]) + dinv * dinv * t_ref[...]
                             + b_ref[...], 0.0)
    giota = lax.broadcasted_iota(jnp.int32, (1, 80), 1)
    ind = jnp.where(batch_ref[...] < giota, 1.0, 0.0)
    starts_ref[...] = jnp.sum(ind, axis=0, keepdims=True).astype(jnp.int32)


def _k3a(parts, t3, dinv, b3, batch2d):
    return pl.pallas_call(
        _k3a_body,
        out_shape=(
            jax.ShapeDtypeStruct((N, H), jnp.float32),
            jax.ShapeDtypeStruct((1, 80), jnp.int32),
        ),
    )(parts, t3, dinv, b3, batch2d)


def _k3b_body(pooled_ref, wl_ref, bl_ref, out_ref):
    out_ref[...] = _mm(pooled_ref[...], wl_ref[...]) + bl_ref[...]


def _k3b(pooled, wl_pad, bl_pad):
    return pl.pallas_call(
        _k3b_body,
        out_shape=jax.ShapeDtypeStruct((G, 16), jnp.float32),
    )(pooled, wl_pad, bl_pad)


def kernel(x, edge_index, edge_weight, batch,
           W1, b1, W2, b2, W3, b3, Wl, bl):
    src1d = edge_index[0]
    dst1d = edge_index[1]
    ew = edge_weight.reshape(-1)
    ew2d_tc = ew.reshape(E // 128, 128)
    zrows = jnp.zeros((ROWS_PER_TILE, H), jnp.float32)
    batch2d = batch.reshape(N, 1)
    b1r = b1.reshape(1, H)
    b2r = b2.reshape(1, H)
    b3r = b3.reshape(1, H)
    wl_pad = jnp.zeros((16, D), jnp.float32).at[:C].set(Wl)
    bl_pad = jnp.zeros((1, 16), jnp.float32).at[0, :C].set(bl)

    pdeg_full = _sc_deg(dst1d, ew, zrows)
    pdeg = pdeg_full[:, :, 0:1]

    wn2d_tc, dinv, t1, hp1 = _k1(ew2d_tc, pdeg, x, W1)
    wn1d = wn2d_tc.reshape(-1)

    parts1 = _sc_agg(hp1, src1d, dst1d, wn1d, zrows)
    t2, hp2 = _k2(parts1, t1, dinv, b1r, W2)
    parts2 = _sc_agg(hp2, src1d, dst1d, wn1d, zrows)
    t3, hp3 = _k2(parts2, t2, dinv, b2r, W3)
    parts3 = _sc_agg(hp3, src1d, dst1d, wn1d, zrows)
    h3, starts = _k3a(parts3, t3, dinv, b3r, batch2d)
    pooled = _sc_pool(h3, starts.reshape(-1), zrows).reshape(G, H)
    out16 = _k3b(pooled, wl_pad, bl_pad)
    return out16[:, :C]
